# baseline (speedup 1.0000x reference)
import jax
import jax.numpy as jnp
from jax.experimental import pallas as pl
from jax.experimental.pallas import tpu as pltpu

BM = 512
BK = 512
BR = 512


def _split(v):
    hi = v.astype(jnp.bfloat16)
    lo = (v - hi.astype(jnp.float32)).astype(jnp.bfloat16)
    return hi, lo


def _prep1_kernel(d_ref, x_ref, xh_ref, xl_ref, rs_ref):
    deg = jnp.max(d_ref[...], axis=1, keepdims=True)
    rs = jax.lax.rsqrt(deg)
    hi, lo = _split(x_ref[...] * rs)
    xh_ref[...] = hi
    xl_ref[...] = lo
    rs_ref[...] = rs


def _prep2_kernel(a_ref, c_ref, rs_ref, mb_ref, rsdj0_ref):
    a = a_ref[...]
    mb_ref[...] = a.astype(jnp.bfloat16)
    val = a * c_ref[...]
    mx = jnp.max(val, axis=1, keepdims=True)
    onehot = (val == mx).astype(jnp.float32)
    rsdj0_ref[...] = jnp.dot(onehot, rs_ref[...],
                             preferred_element_type=jnp.float32)


def _layer1_kernel(mb_ref, xh_ref, xl_ref, rsdj0_ref, rsi_ref, w1t_ref, b1_ref,
                   hh_ref, hl_ref, acc):
    k = pl.program_id(1)
    nk = pl.num_programs(1)

    @pl.when(k == 0)
    def _init():
        acc[...] = jnp.zeros_like(acc)

    mb = mb_ref[...]
    kb = mb.shape[1]
    xh = xh_ref[pl.ds(k * kb, kb), :]
    xl = xl_ref[pl.ds(k * kb, kb), :]
    acc[...] += (jnp.dot(mb, xh, preferred_element_type=jnp.float32)
                 + jnp.dot(mb, xl, preferred_element_type=jnp.float32))

    @pl.when(k == nk - 1)
    def _epilogue():
        agg = acc[...] * rsdj0_ref[...]
        h = jnp.dot(agg, w1t_ref[...], preferred_element_type=jnp.float32)
        h = h + b1_ref[...]
        h = jnp.where(h > 0, h, 0.01 * h)
        hi, lo = _split(h * rsi_ref[...])
        hh_ref[...] = hi
        hl_ref[...] = lo


def _layer2_kernel(mb_ref, hh_ref, hl_ref, rsdj0_ref, w2t_ref, b2_ref,
                   w3t_ref, b3_ref, out_ref, acc):
    k = pl.program_id(1)
    nk = pl.num_programs(1)

    @pl.when(k == 0)
    def _init():
        acc[...] = jnp.zeros_like(acc)

    mb = mb_ref[...]
    kb = mb.shape[1]
    hh = hh_ref[pl.ds(k * kb, kb), :]
    hl = hl_ref[pl.ds(k * kb, kb), :]
    acc[...] += (jnp.dot(mb, hh, preferred_element_type=jnp.float32)
                 + jnp.dot(mb, hl, preferred_element_type=jnp.float32))

    @pl.when(k == nk - 1)
    def _epilogue():
        agg = acc[...] * rsdj0_ref[...]
        h = jnp.dot(agg, w2t_ref[...], preferred_element_type=jnp.float32)
        h = h + b2_ref[...]
        h = jnp.where(h > 0, h, 0.01 * h)
        o = jnp.dot(h, w3t_ref[...], preferred_element_type=jnp.float32)
        o = o + b3_ref[...]
        m = jnp.max(o, axis=1, keepdims=True)
        e = jnp.exp(o - m)
        out_ref[...] = (o - m) - jnp.log(jnp.sum(e, axis=1, keepdims=True))


def kernel(D, X, A, W1, b1, W2, b2, W3, b3):
    n, f = X.shape
    h1 = W1.shape[0]
    h2 = W2.shape[0]
    c = W3.shape[0]
    f32 = jnp.float32
    bf16 = jnp.bfloat16

    xh, xl, rsdeg = pl.pallas_call(
        _prep1_kernel,
        grid=(n // BR,),
        in_specs=[
            pl.BlockSpec((BR, n), lambda i: (i, 0)),
            pl.BlockSpec((BR, f), lambda i: (i, 0)),
        ],
        out_specs=[
            pl.BlockSpec((BR, f), lambda i: (i, 0)),
            pl.BlockSpec((BR, f), lambda i: (i, 0)),
            pl.BlockSpec((BR, 1), lambda i: (i, 0)),
        ],
        out_shape=[
            jax.ShapeDtypeStruct((n, f), bf16),
            jax.ShapeDtypeStruct((n, f), bf16),
            jax.ShapeDtypeStruct((n, 1), f32),
        ],
    )(D, X)

    cvec = (n - jnp.arange(n, dtype=jnp.int32)).reshape(1, n)

    mb, rsdj0 = pl.pallas_call(
        _prep2_kernel,
        grid=(n // BR,),
        in_specs=[
            pl.BlockSpec((BR, n), lambda i: (i, 0)),
            pl.BlockSpec((1, n), lambda i: (0, 0)),
            pl.BlockSpec((n, 1), lambda i: (0, 0)),
        ],
        out_specs=[
            pl.BlockSpec((BR, n), lambda i: (i, 0)),
            pl.BlockSpec((BR, 1), lambda i: (i, 0)),
        ],
        out_shape=[
            jax.ShapeDtypeStruct((n, n), bf16),
            jax.ShapeDtypeStruct((n, 1), f32),
        ],
    )(A, cvec, rsdeg)

    w1t = jnp.transpose(W1)
    w2t = jnp.transpose(W2)
    w3t = jnp.transpose(W3)
    b1r = jnp.reshape(b1, (1, h1))
    b2r = jnp.reshape(b2, (1, h2))
    b3r = jnp.reshape(b3, (1, c))

    hh, hl = pl.pallas_call(
        _layer1_kernel,
        grid=(n // BM, n // BK),
        in_specs=[
            pl.BlockSpec((BM, BK), lambda i, k: (i, k)),
            pl.BlockSpec((n, f), lambda i, k: (0, 0)),
            pl.BlockSpec((n, f), lambda i, k: (0, 0)),
            pl.BlockSpec((BM, 1), lambda i, k: (i, 0)),
            pl.BlockSpec((BM, 1), lambda i, k: (i, 0)),
            pl.BlockSpec((f, h1), lambda i, k: (0, 0)),
            pl.BlockSpec((1, h1), lambda i, k: (0, 0)),
        ],
        out_specs=[
            pl.BlockSpec((BM, h1), lambda i, k: (i, 0)),
            pl.BlockSpec((BM, h1), lambda i, k: (i, 0)),
        ],
        out_shape=[
            jax.ShapeDtypeStruct((n, h1), bf16),
            jax.ShapeDtypeStruct((n, h1), bf16),
        ],
        scratch_shapes=[pltpu.VMEM((BM, f), f32)],
    )(mb, xh, xl, rsdj0, rsdeg, w1t, b1r)

    out = pl.pallas_call(
        _layer2_kernel,
        grid=(n // BM, n // BK),
        in_specs=[
            pl.BlockSpec((BM, BK), lambda i, k: (i, k)),
            pl.BlockSpec((n, h1), lambda i, k: (0, 0)),
            pl.BlockSpec((n, h1), lambda i, k: (0, 0)),
            pl.BlockSpec((BM, 1), lambda i, k: (i, 0)),
            pl.BlockSpec((h1, h2), lambda i, k: (0, 0)),
            pl.BlockSpec((1, h2), lambda i, k: (0, 0)),
            pl.BlockSpec((h2, c), lambda i, k: (0, 0)),
            pl.BlockSpec((1, c), lambda i, k: (0, 0)),
        ],
        out_specs=pl.BlockSpec((BM, c), lambda i, k: (i, 0)),
        out_shape=jax.ShapeDtypeStruct((n, c), f32),
        scratch_shapes=[pltpu.VMEM((BM, h1), f32)],
    )(mb, hh, hl, rsdj0, w2t, b2r, w3t, b3r)

    return out


# speedup vs baseline: 1.5274x; 1.5274x over previous
import jax
import jax.numpy as jnp
from jax.experimental import pallas as pl
from jax.experimental.pallas import tpu as pltpu

BM = 512
BK = 512
BR = 512


def _split(v):
    hi = v.astype(jnp.bfloat16)
    lo = (v - hi.astype(jnp.float32)).astype(jnp.bfloat16)
    return hi, lo


def _prep1_kernel(d_ref, x_ref, xh_ref, xl_ref, rs_ref):
    deg = jnp.max(d_ref[...], axis=1, keepdims=True)
    rs = jax.lax.rsqrt(deg)
    hi, lo = _split(x_ref[...] * rs)
    xh_ref[...] = hi
    xl_ref[...] = lo
    rs_ref[...] = rs


def _prep2_kernel(a_ref, c_ref, rs_ref, mb_ref, rsdj0_ref):
    a = a_ref[...]
    mb_ref[...] = a.astype(jnp.bfloat16)
    val = a * c_ref[...]
    mx = jnp.max(val, axis=1, keepdims=True)
    onehot = (val == mx).astype(jnp.float32)
    rsdj0_ref[...] = jnp.dot(onehot, rs_ref[...],
                             preferred_element_type=jnp.float32)


def _layer1_kernel(mb_ref, xh_ref, xl_ref, rsdj0_ref, rsi_ref, w1t_ref, b1_ref,
                   hh_ref, hl_ref):
    mb = mb_ref[...]
    acc = (jnp.dot(mb, xh_ref[...], preferred_element_type=jnp.float32)
           + jnp.dot(mb, xl_ref[...], preferred_element_type=jnp.float32))
    agg = acc * rsdj0_ref[...]
    h = jnp.dot(agg, w1t_ref[...], preferred_element_type=jnp.float32)
    h = h + b1_ref[...]
    h = jnp.where(h > 0, h, 0.01 * h)
    hi, lo = _split(h * rsi_ref[...])
    hh_ref[...] = hi
    hl_ref[...] = lo


def _layer2_kernel(mb_ref, hh_ref, hl_ref, rsdj0_ref, w2t_ref, b2_ref,
                   w3t_ref, b3_ref, out_ref):
    mb = mb_ref[...]
    acc = (jnp.dot(mb, hh_ref[...], preferred_element_type=jnp.float32)
           + jnp.dot(mb, hl_ref[...], preferred_element_type=jnp.float32))
    agg = acc * rsdj0_ref[...]
    h = jnp.dot(agg, w2t_ref[...], preferred_element_type=jnp.float32)
    h = h + b2_ref[...]
    h = jnp.where(h > 0, h, 0.01 * h)
    o = jnp.dot(h, w3t_ref[...], preferred_element_type=jnp.float32)
    o = o + b3_ref[...]
    m = jnp.max(o, axis=1, keepdims=True)
    e = jnp.exp(o - m)
    out_ref[...] = (o - m) - jnp.log(jnp.sum(e, axis=1, keepdims=True))


def kernel(D, X, A, W1, b1, W2, b2, W3, b3):
    n, f = X.shape
    h1 = W1.shape[0]
    h2 = W2.shape[0]
    c = W3.shape[0]
    f32 = jnp.float32
    bf16 = jnp.bfloat16

    xh, xl, rsdeg = pl.pallas_call(
        _prep1_kernel,
        grid=(n // BR,),
        in_specs=[
            pl.BlockSpec((BR, n), lambda i: (i, 0)),
            pl.BlockSpec((BR, f), lambda i: (i, 0)),
        ],
        out_specs=[
            pl.BlockSpec((BR, f), lambda i: (i, 0)),
            pl.BlockSpec((BR, f), lambda i: (i, 0)),
            pl.BlockSpec((BR, 1), lambda i: (i, 0)),
        ],
        out_shape=[
            jax.ShapeDtypeStruct((n, f), bf16),
            jax.ShapeDtypeStruct((n, f), bf16),
            jax.ShapeDtypeStruct((n, 1), f32),
        ],
    )(D, X)

    cvec = (n - jnp.arange(n, dtype=jnp.int32)).reshape(1, n)

    mb, rsdj0 = pl.pallas_call(
        _prep2_kernel,
        grid=(n // BR,),
        in_specs=[
            pl.BlockSpec((BR, n), lambda i: (i, 0)),
            pl.BlockSpec((1, n), lambda i: (0, 0)),
            pl.BlockSpec((n, 1), lambda i: (0, 0)),
        ],
        out_specs=[
            pl.BlockSpec((BR, n), lambda i: (i, 0)),
            pl.BlockSpec((BR, 1), lambda i: (i, 0)),
        ],
        out_shape=[
            jax.ShapeDtypeStruct((n, n), bf16),
            jax.ShapeDtypeStruct((n, 1), f32),
        ],
    )(A, cvec, rsdeg)

    w1t = jnp.transpose(W1)
    w2t = jnp.transpose(W2)
    w3t = jnp.transpose(W3)
    b1r = jnp.reshape(b1, (1, h1))
    b2r = jnp.reshape(b2, (1, h2))
    b3r = jnp.reshape(b3, (1, c))

    hh, hl = pl.pallas_call(
        _layer1_kernel,
        grid=(n // BM,),
        in_specs=[
            pl.BlockSpec((BM, n), lambda i: (i, 0)),
            pl.BlockSpec((n, f), lambda i: (0, 0)),
            pl.BlockSpec((n, f), lambda i: (0, 0)),
            pl.BlockSpec((BM, 1), lambda i: (i, 0)),
            pl.BlockSpec((BM, 1), lambda i: (i, 0)),
            pl.BlockSpec((f, h1), lambda i: (0, 0)),
            pl.BlockSpec((1, h1), lambda i: (0, 0)),
        ],
        out_specs=[
            pl.BlockSpec((BM, h1), lambda i: (i, 0)),
            pl.BlockSpec((BM, h1), lambda i: (i, 0)),
        ],
        out_shape=[
            jax.ShapeDtypeStruct((n, h1), bf16),
            jax.ShapeDtypeStruct((n, h1), bf16),
        ],
    )(mb, xh, xl, rsdj0, rsdeg, w1t, b1r)

    out = pl.pallas_call(
        _layer2_kernel,
        grid=(n // BM,),
        in_specs=[
            pl.BlockSpec((BM, n), lambda i: (i, 0)),
            pl.BlockSpec((n, h1), lambda i: (0, 0)),
            pl.BlockSpec((n, h1), lambda i: (0, 0)),
            pl.BlockSpec((BM, 1), lambda i: (i, 0)),
            pl.BlockSpec((h1, h2), lambda i: (0, 0)),
            pl.BlockSpec((1, h2), lambda i: (0, 0)),
            pl.BlockSpec((h2, c), lambda i: (0, 0)),
            pl.BlockSpec((1, c), lambda i: (0, 0)),
        ],
        out_specs=pl.BlockSpec((BM, c), lambda i: (i, 0)),
        out_shape=jax.ShapeDtypeStruct((n, c), f32),
    )(mb, hh, hl, rsdj0, w2t, b2r, w3t, b3r)

    return out


# speedup vs baseline: 1.7937x; 1.1743x over previous
import jax
import jax.numpy as jnp
from jax.experimental import pallas as pl
from jax.experimental.pallas import tpu as pltpu

BM = 512
BK = 512
BR = 512


def _split(v):
    hi = v.astype(jnp.bfloat16)
    lo = (v - hi.astype(jnp.float32)).astype(jnp.bfloat16)
    return hi, lo


def _prep1_kernel(d_ref, x_ref, xh_ref, xl_ref, rs_ref):
    deg = jnp.max(d_ref[...], axis=1, keepdims=True)
    rs = jax.lax.rsqrt(deg)
    hi, lo = _split(x_ref[...] * rs)
    xh_ref[...] = hi
    xl_ref[...] = lo
    rs_ref[...] = rs


def _layer1_kernel(a_ref, c_ref, rs_ref, xh_ref, xl_ref, rsi_ref, w1t_ref,
                   b1_ref, mb_ref, rsdj0_ref, hh_ref, hl_ref):
    a = a_ref[...]
    mb = a.astype(jnp.bfloat16)
    mb_ref[...] = mb
    val = a * c_ref[...]
    mx = jnp.max(val, axis=1, keepdims=True)
    onehot = (val == mx).astype(jnp.float32)
    rsdj0 = jnp.dot(onehot, rs_ref[...], preferred_element_type=jnp.float32)
    rsdj0_ref[...] = rsdj0

    acc = (jnp.dot(mb, xh_ref[...], preferred_element_type=jnp.float32)
           + jnp.dot(mb, xl_ref[...], preferred_element_type=jnp.float32))
    agg = acc * rsdj0
    h = jnp.dot(agg, w1t_ref[...], preferred_element_type=jnp.float32)
    h = h + b1_ref[...]
    h = jnp.where(h > 0, h, 0.01 * h)
    hi, lo = _split(h * rsi_ref[...])
    hh_ref[...] = hi
    hl_ref[...] = lo


def _layer2_kernel(mb_ref, hh_ref, hl_ref, rsdj0_ref, w2t_ref, b2_ref,
                   w3t_ref, b3_ref, out_ref):
    mb = mb_ref[...]
    acc = (jnp.dot(mb, hh_ref[...], preferred_element_type=jnp.float32)
           + jnp.dot(mb, hl_ref[...], preferred_element_type=jnp.float32))
    agg = acc * rsdj0_ref[...]
    h = jnp.dot(agg, w2t_ref[...], preferred_element_type=jnp.float32)
    h = h + b2_ref[...]
    h = jnp.where(h > 0, h, 0.01 * h)
    o = jnp.dot(h, w3t_ref[...], preferred_element_type=jnp.float32)
    o = o + b3_ref[...]
    m = jnp.max(o, axis=1, keepdims=True)
    e = jnp.exp(o - m)
    out_ref[...] = (o - m) - jnp.log(jnp.sum(e, axis=1, keepdims=True))


def kernel(D, X, A, W1, b1, W2, b2, W3, b3):
    n, f = X.shape
    h1 = W1.shape[0]
    h2 = W2.shape[0]
    c = W3.shape[0]
    f32 = jnp.float32
    bf16 = jnp.bfloat16

    xh, xl, rsdeg = pl.pallas_call(
        _prep1_kernel,
        grid=(n // BR,),
        in_specs=[
            pl.BlockSpec((BR, n), lambda i: (i, 0)),
            pl.BlockSpec((BR, f), lambda i: (i, 0)),
        ],
        out_specs=[
            pl.BlockSpec((BR, f), lambda i: (i, 0)),
            pl.BlockSpec((BR, f), lambda i: (i, 0)),
            pl.BlockSpec((BR, 1), lambda i: (i, 0)),
        ],
        out_shape=[
            jax.ShapeDtypeStruct((n, f), bf16),
            jax.ShapeDtypeStruct((n, f), bf16),
            jax.ShapeDtypeStruct((n, 1), f32),
        ],
    )(D, X)

    cvec = (n - jnp.arange(n, dtype=jnp.int32)).reshape(1, n)

    w1t = jnp.transpose(W1)
    w2t = jnp.transpose(W2)
    w3t = jnp.transpose(W3)
    b1r = jnp.reshape(b1, (1, h1))
    b2r = jnp.reshape(b2, (1, h2))
    b3r = jnp.reshape(b3, (1, c))

    mb, rsdj0, hh, hl = pl.pallas_call(
        _layer1_kernel,
        grid=(n // BM,),
        in_specs=[
            pl.BlockSpec((BM, n), lambda i: (i, 0)),
            pl.BlockSpec((1, n), lambda i: (0, 0)),
            pl.BlockSpec((n, 1), lambda i: (0, 0)),
            pl.BlockSpec((n, f), lambda i: (0, 0)),
            pl.BlockSpec((n, f), lambda i: (0, 0)),
            pl.BlockSpec((BM, 1), lambda i: (i, 0)),
            pl.BlockSpec((f, h1), lambda i: (0, 0)),
            pl.BlockSpec((1, h1), lambda i: (0, 0)),
        ],
        out_specs=[
            pl.BlockSpec((BM, n), lambda i: (i, 0)),
            pl.BlockSpec((BM, 1), lambda i: (i, 0)),
            pl.BlockSpec((BM, h1), lambda i: (i, 0)),
            pl.BlockSpec((BM, h1), lambda i: (i, 0)),
        ],
        out_shape=[
            jax.ShapeDtypeStruct((n, n), bf16),
            jax.ShapeDtypeStruct((n, 1), f32),
            jax.ShapeDtypeStruct((n, h1), bf16),
            jax.ShapeDtypeStruct((n, h1), bf16),
        ],
    )(A, cvec, rsdeg, xh, xl, rsdeg, w1t, b1r)

    out = pl.pallas_call(
        _layer2_kernel,
        grid=(n // BM,),
        in_specs=[
            pl.BlockSpec((BM, n), lambda i: (i, 0)),
            pl.BlockSpec((n, h1), lambda i: (0, 0)),
            pl.BlockSpec((n, h1), lambda i: (0, 0)),
            pl.BlockSpec((BM, 1), lambda i: (i, 0)),
            pl.BlockSpec((h1, h2), lambda i: (0, 0)),
            pl.BlockSpec((1, h2), lambda i: (0, 0)),
            pl.BlockSpec((h2, c), lambda i: (0, 0)),
            pl.BlockSpec((1, c), lambda i: (0, 0)),
        ],
        out_specs=pl.BlockSpec((BM, c), lambda i: (i, 0)),
        out_shape=jax.ShapeDtypeStruct((n, c), f32),
    )(mb, hh, hl, rsdj0, w2t, b2r, w3t, b3r)

    return out


# speedup vs baseline: 2.1083x; 1.1754x over previous
import jax
import jax.numpy as jnp
from jax.experimental import pallas as pl
from jax.experimental.pallas import tpu as pltpu

BM = 512
BR = 512
LG = 128


def _split(v):
    hi = v.astype(jnp.bfloat16)
    lo = (v - hi.astype(jnp.float32)).astype(jnp.bfloat16)
    return hi, lo


def _prep_kernel(d_ref, x_ref, xh_ref, xl_ref, rs_ref):
    deg = jnp.max(d_ref[...], axis=1, keepdims=True)
    rs = jax.lax.rsqrt(deg)
    hi, lo = _split(x_ref[...] * rs)
    xh_ref[...] = hi
    xl_ref[...] = lo
    rs_ref[...] = rs


def _gcn_kernel(a_ref, c_ref, rs_ref, rs2_ref, xh_ref, xl_ref, w1t_ref, b1_ref,
                w2t_ref, b2_ref, w3t_ref, b3_ref, out_ref,
                mask_scr, hh_scr, rsdj0_scr):
    p = pl.program_id(0)
    i = pl.program_id(1)
    n = mask_scr.shape[1]
    bm = a_ref.shape[0]

    @pl.when(p == 0)
    def _layer1():
        a = a_ref[...]
        mb = a.astype(jnp.bfloat16)
        mask_scr[pl.ds(i * bm, bm), :] = mb
        val = a * c_ref[...]
        idx = n - jnp.max(val, axis=1, keepdims=True)
        q = idx // LG
        r = idx - q * LG
        ohq = (jax.lax.broadcasted_iota(jnp.int32, (bm, n // LG), 1) == q
               ).astype(jnp.float32)
        ohr = (jax.lax.broadcasted_iota(jnp.int32, (bm, LG), 1) == r
               ).astype(jnp.float32)
        rsq = jnp.dot(ohq, rs2_ref[...], preferred_element_type=jnp.float32)
        rsdj0 = jnp.sum(rsq * ohr, axis=1, keepdims=True)
        rsdj0_scr[pl.ds(i * bm, bm), :] = rsdj0

        acc = (jnp.dot(mb, xh_ref[...], preferred_element_type=jnp.float32)
               + jnp.dot(mb, xl_ref[...], preferred_element_type=jnp.float32))
        agg = acc * rsdj0
        h = jnp.dot(agg, w1t_ref[...], preferred_element_type=jnp.float32)
        h = h + b1_ref[...]
        h = jnp.where(h > 0, h, 0.01 * h)
        rsi = rs_ref[pl.ds(i * bm, bm), :]
        hh_scr[pl.ds(i * bm, bm), :] = (h * rsi).astype(jnp.bfloat16)

    @pl.when(p == 1)
    def _layer2():
        mb = mask_scr[pl.ds(i * bm, bm), :]
        acc = jnp.dot(mb, hh_scr[...], preferred_element_type=jnp.float32)
        agg = acc * rsdj0_scr[pl.ds(i * bm, bm), :]
        h = jnp.dot(agg, w2t_ref[...], preferred_element_type=jnp.float32)
        h = h + b2_ref[...]
        h = jnp.where(h > 0, h, 0.01 * h)
        o = jnp.dot(h, w3t_ref[...], preferred_element_type=jnp.float32)
        o = o + b3_ref[...]
        m = jnp.max(o, axis=1, keepdims=True)
        e = jnp.exp(o - m)
        out_ref[...] = (o - m) - jnp.log(jnp.sum(e, axis=1, keepdims=True))


def kernel(D, X, A, W1, b1, W2, b2, W3, b3):
    n, f = X.shape
    h1 = W1.shape[0]
    h2 = W2.shape[0]
    c = W3.shape[0]
    f32 = jnp.float32
    bf16 = jnp.bfloat16

    xh, xl, rsdeg = pl.pallas_call(
        _prep_kernel,
        grid=(n // BR,),
        in_specs=[
            pl.BlockSpec((BR, n), lambda i: (i, 0)),
            pl.BlockSpec((BR, f), lambda i: (i, 0)),
        ],
        out_specs=[
            pl.BlockSpec((BR, f), lambda i: (i, 0)),
            pl.BlockSpec((BR, f), lambda i: (i, 0)),
            pl.BlockSpec((BR, 1), lambda i: (i, 0)),
        ],
        out_shape=[
            jax.ShapeDtypeStruct((n, f), bf16),
            jax.ShapeDtypeStruct((n, f), bf16),
            jax.ShapeDtypeStruct((n, 1), f32),
        ],
    )(D, X)

    cvec = (n - jnp.arange(n, dtype=jnp.int32)).reshape(1, n)
    rs2 = jnp.reshape(rsdeg, (n // LG, LG))

    w1t = jnp.transpose(W1)
    w2t = jnp.transpose(W2)
    w3t = jnp.transpose(W3)
    b1r = jnp.reshape(b1, (1, h1))
    b2r = jnp.reshape(b2, (1, h2))
    b3r = jnp.reshape(b3, (1, c))

    ns = n // BM
    out = pl.pallas_call(
        _gcn_kernel,
        grid=(2, ns),
        in_specs=[
            pl.BlockSpec((BM, n), lambda p, i: (jnp.where(p == 0, i, ns - 1), 0)),
            pl.BlockSpec((1, n), lambda p, i: (0, 0)),
            pl.BlockSpec((n, 1), lambda p, i: (0, 0)),
            pl.BlockSpec((n // LG, LG), lambda p, i: (0, 0)),
            pl.BlockSpec((n, f), lambda p, i: (0, 0)),
            pl.BlockSpec((n, f), lambda p, i: (0, 0)),
            pl.BlockSpec((f, h1), lambda p, i: (0, 0)),
            pl.BlockSpec((1, h1), lambda p, i: (0, 0)),
            pl.BlockSpec((h1, h2), lambda p, i: (0, 0)),
            pl.BlockSpec((1, h2), lambda p, i: (0, 0)),
            pl.BlockSpec((h2, c), lambda p, i: (0, 0)),
            pl.BlockSpec((1, c), lambda p, i: (0, 0)),
        ],
        out_specs=pl.BlockSpec((BM, c), lambda p, i: (i, 0)),
        out_shape=jax.ShapeDtypeStruct((n, c), f32),
        scratch_shapes=[
            pltpu.VMEM((n, n), bf16),
            pltpu.VMEM((n, h1), bf16),
            pltpu.VMEM((n, 1), f32),
        ],
    )(A, cvec, rsdeg, rs2, xh, xl, w1t, b1r, w2t, b2r, w3t, b3r)

    return out
